# Initial kernel scaffold; baseline (speedup 1.0000x reference)
import jax
import jax.numpy as jnp
from jax.experimental import pallas as pl


def kernel(standardized_hidden, W_enc, b_enc, W_dec, b_dec):
    raise NotImplementedError("write your pallas kernel here")


# speedup vs baseline: 8.9809x; 8.9809x over previous
import jax
import jax.numpy as jnp
from jax.experimental import pallas as pl

D_MODEL = 768
WIDTH = 8192
TOP_K = 32
T_ENC = 128
T_DEC = 256


def _encode_kernel(x_ref, we_ref, be_ref, feat_ref):
    x = x_ref[...]
    dense = jax.lax.dot_general(
        x, we_ref[...], (((1,), (1,)), ((), ())),
        preferred_element_type=jnp.float32,
    )
    dense = jnp.maximum(dense + be_ref[...], 0.0)

    bits = dense.view(jnp.int32)
    lo = jnp.zeros((T_ENC, 1), jnp.int32)
    hi = jnp.max(bits, axis=1, keepdims=True) + 1

    def body(_, carry):
        lo, hi = carry
        mid = lo + (hi - lo) // 2
        cnt = jnp.sum((bits >= mid).astype(jnp.int32), axis=1, keepdims=True)
        ge = cnt >= TOP_K
        return jnp.where(ge, mid, lo), jnp.where(ge, hi, mid)

    lo, hi = jax.lax.fori_loop(0, 31, body, (lo, hi))
    feat_ref[...] = jnp.where(bits >= lo, dense, 0.0)


def _decode_kernel(feat_ref, wdt_ref, bd_ref, delta_ref):
    delta = jnp.dot(
        feat_ref[...], wdt_ref[...],
        preferred_element_type=jnp.float32,
    )
    delta_ref[...] = delta + bd_ref[...]


@jax.jit
def kernel(standardized_hidden, W_enc, b_enc, W_dec, b_dec):
    B, S, D = standardized_hidden.shape
    x = standardized_hidden.reshape(B * S, D)
    n_tok = B * S

    feat = pl.pallas_call(
        _encode_kernel,
        grid=(n_tok // T_ENC,),
        in_specs=[
            pl.BlockSpec((T_ENC, D_MODEL), lambda i: (i, 0)),
            pl.BlockSpec((WIDTH, D_MODEL), lambda i: (0, 0)),
            pl.BlockSpec((1, WIDTH), lambda i: (0, 0)),
        ],
        out_specs=pl.BlockSpec((T_ENC, WIDTH), lambda i: (i, 0)),
        out_shape=jax.ShapeDtypeStruct((n_tok, WIDTH), jnp.float32),
    )(x, W_enc, b_enc.reshape(1, WIDTH))

    delta = pl.pallas_call(
        _decode_kernel,
        grid=(n_tok // T_DEC,),
        in_specs=[
            pl.BlockSpec((T_DEC, WIDTH), lambda i: (i, 0)),
            pl.BlockSpec((WIDTH, D_MODEL), lambda i: (0, 0)),
            pl.BlockSpec((1, D_MODEL), lambda i: (0, 0)),
        ],
        out_specs=pl.BlockSpec((T_DEC, D_MODEL), lambda i: (i, 0)),
        out_shape=jax.ShapeDtypeStruct((n_tok, D_MODEL), jnp.float32),
    )(feat, W_dec.T, b_dec.reshape(1, D_MODEL))

    return (delta.reshape(B, S, D), feat.reshape(B, S, WIDTH))


# speedup vs baseline: 11.4454x; 1.2744x over previous
import jax
import jax.numpy as jnp
from jax.experimental import pallas as pl

D_MODEL = 768
WIDTH = 8192
TOP_K = 32
T_ENC = 128
T_DEC = 256


def _encode_kernel(x_ref, we_ref, be_ref, feat_ref):
    x = x_ref[...]
    dense = jax.lax.dot_general(
        x, we_ref[...], (((1,), (1,)), ((), ())),
        preferred_element_type=jnp.float32,
    )
    dense = jnp.maximum(dense + be_ref[...], 0.0)

    bits = dense.view(jnp.int32)
    lo = jnp.zeros((T_ENC, 1), jnp.int32)
    hi = jnp.max(bits, axis=1, keepdims=True) + 1

    def cond(carry):
        lo, hi = carry
        return jnp.any(hi - lo > 1)

    def body(carry):
        lo, hi = carry
        mid = lo + (hi - lo) // 2
        cnt = jnp.sum((bits >= mid).astype(jnp.int32), axis=1, keepdims=True)
        ge = cnt >= TOP_K
        eq = cnt == TOP_K
        return (jnp.where(ge, mid, lo),
                jnp.where(eq, mid + 1, jnp.where(ge, hi, mid)))

    lo, hi = jax.lax.while_loop(cond, body, (lo, hi))
    feat_ref[...] = jnp.where(bits >= lo, dense, 0.0)


def _decode_kernel(feat_ref, wd_ref, bd_ref, delta_ref):
    delta = jax.lax.dot_general(
        feat_ref[...], wd_ref[...], (((1,), (1,)), ((), ())),
        preferred_element_type=jnp.float32,
    )
    delta_ref[...] = delta + bd_ref[...]


@jax.jit
def kernel(standardized_hidden, W_enc, b_enc, W_dec, b_dec):
    B, S, D = standardized_hidden.shape
    x = standardized_hidden.reshape(B * S, D)
    n_tok = B * S

    feat = pl.pallas_call(
        _encode_kernel,
        grid=(n_tok // T_ENC,),
        in_specs=[
            pl.BlockSpec((T_ENC, D_MODEL), lambda i: (i, 0)),
            pl.BlockSpec((WIDTH, D_MODEL), lambda i: (0, 0)),
            pl.BlockSpec((1, WIDTH), lambda i: (0, 0)),
        ],
        out_specs=pl.BlockSpec((T_ENC, WIDTH), lambda i: (i, 0)),
        out_shape=jax.ShapeDtypeStruct((n_tok, WIDTH), jnp.float32),
    )(x, W_enc, b_enc.reshape(1, WIDTH))

    delta = pl.pallas_call(
        _decode_kernel,
        grid=(n_tok // T_DEC,),
        in_specs=[
            pl.BlockSpec((T_DEC, WIDTH), lambda i: (i, 0)),
            pl.BlockSpec((D_MODEL, WIDTH), lambda i: (0, 0)),
            pl.BlockSpec((1, D_MODEL), lambda i: (0, 0)),
        ],
        out_specs=pl.BlockSpec((T_DEC, D_MODEL), lambda i: (i, 0)),
        out_shape=jax.ShapeDtypeStruct((n_tok, D_MODEL), jnp.float32),
    )(feat, W_dec, b_dec.reshape(1, D_MODEL))

    return (delta.reshape(B, S, D), feat.reshape(B, S, WIDTH))
